# Initial kernel scaffold; baseline (speedup 1.0000x reference)
import functools

import jax
import jax.numpy as jnp
from jax import lax
from jax.experimental import pallas as pl
from jax.experimental.pallas import tpu as pltpu
from jax.experimental.pallas import tpu_sc as plsc

N = 10000
D = 128
H = 128
NC = 2
NS = 16
NW = NC * NS
LANE = 128
Np = 10240
RPT = Np // NS
ROWBLK = 1000

_mesh = plsc.VectorSubcoreMesh(
    core_axis_name="c", subcore_axis_name="s", num_cores=NC, num_subcores=NS)


def _make_degrees(ch):
    def body(srcd, dstd, degp, src_v, dst_v, ones_v, zbuf, acc_out, acc_in):
        c = lax.axis_index("c")
        s = lax.axis_index("s")
        wid = c * NS + s
        pltpu.sync_copy(srcd.at[pl.ds(wid * ch, ch)], src_v)
        pltpu.sync_copy(dstd.at[pl.ds(wid * ch, ch)], dst_v)
        for i in range(LANE // 16):
            ones_v[pl.ds(i * 16, 16)] = jnp.ones((16,), jnp.float32)
        for i in range(RPT // 16):
            zbuf[pl.ds(i * 16, 16)] = jnp.zeros((16,), jnp.float32)
        pltpu.sync_copy(zbuf, acc_out.at[pl.ds(s * RPT, RPT)])
        pltpu.sync_copy(zbuf, acc_in.at[pl.ds(s * RPT, RPT)])
        plsc.subcore_barrier()

        def step(j, carry):
            pltpu.sync_copy(ones_v, acc_out.at[src_v.at[j]], add=True)
            pltpu.sync_copy(ones_v, acc_in.at[dst_v.at[j]], add=True)
            return carry

        lax.fori_loop(0, ch, step, 0)
        plsc.subcore_barrier()
        pltpu.sync_copy(acc_out.at[pl.ds(s * RPT, RPT)], zbuf)
        pltpu.sync_copy(zbuf, degp.at[pl.ds((2 * c) * Np + s * RPT, RPT)])
        pltpu.sync_copy(acc_in.at[pl.ds(s * RPT, RPT)], zbuf)
        pltpu.sync_copy(zbuf, degp.at[pl.ds((2 * c + 1) * Np + s * RPT, RPT)])

    return pl.kernel(
        body,
        out_type=jax.ShapeDtypeStruct((4 * Np,), jnp.float32),
        mesh=_mesh,
        scratch_types=[
            pltpu.VMEM((ch, LANE), jnp.int32),
            pltpu.VMEM((ch, LANE), jnp.int32),
            pltpu.VMEM((LANE,), jnp.float32),
            pltpu.VMEM((RPT,), jnp.float32),
            pltpu.VMEM_SHARED((Np,), jnp.float32),
            pltpu.VMEM_SHARED((Np,), jnp.float32),
        ],
    )


def _make_segsum(ch):
    obr = RPT // 2

    def body(t, srcg, dstg, aggp, src_v, dst_v, buf0, buf1, obuf, acc,
             sem0, sem1):
        c = lax.axis_index("c")
        s = lax.axis_index("s")
        wid = c * NS + s
        pltpu.sync_copy(srcg.at[pl.ds(wid * ch, ch)], src_v)
        pltpu.sync_copy(dstg.at[pl.ds(wid * ch, ch)], dst_v)

        def zrow(r, carry):
            for k in range(H // 16):
                obuf[r, pl.ds(k * 16, 16)] = jnp.zeros((16,), jnp.float32)
            return carry

        lax.fori_loop(0, obr, zrow, 0)
        pltpu.sync_copy(obuf, acc.at[pl.ds(s * RPT, obr)])
        pltpu.sync_copy(obuf, acc.at[pl.ds(s * RPT + obr, obr)])
        plsc.subcore_barrier()

        pltpu.async_copy(t.at[src_v.at[0]], buf0, sem0)
        pltpu.async_copy(t.at[src_v.at[1]], buf1, sem1)

        def step(i, carry):
            j = 2 * i
            pltpu.make_async_copy(t.at[src_v.at[j]], buf0, sem0).wait()
            pltpu.sync_copy(buf0, acc.at[dst_v.at[j]], add=True)
            pltpu.async_copy(t.at[src_v.at[j + 2]], buf0, sem0)
            pltpu.make_async_copy(t.at[src_v.at[j + 1]], buf1, sem1).wait()
            pltpu.sync_copy(buf1, acc.at[dst_v.at[j + 1]], add=True)
            pltpu.async_copy(t.at[src_v.at[j + 3]], buf1, sem1)
            return carry

        lax.fori_loop(0, ch // 2 - 1, step, 0)
        jlast = ch - 2
        pltpu.make_async_copy(t.at[src_v.at[jlast]], buf0, sem0).wait()
        pltpu.sync_copy(buf0, acc.at[dst_v.at[jlast]], add=True)
        pltpu.make_async_copy(t.at[src_v.at[jlast + 1]], buf1, sem1).wait()
        pltpu.sync_copy(buf1, acc.at[dst_v.at[jlast + 1]], add=True)

        plsc.subcore_barrier()
        for hh in range(2):
            pltpu.sync_copy(acc.at[pl.ds(s * RPT + hh * obr, obr)], obuf)
            pltpu.sync_copy(
                obuf, aggp.at[pl.ds(c * Np + s * RPT + hh * obr, obr)])

    return pl.kernel(
        body,
        out_type=jax.ShapeDtypeStruct((2 * Np, H), jnp.float32),
        mesh=_mesh,
        scratch_types=[
            pltpu.VMEM((ch, LANE), jnp.int32),
            pltpu.VMEM((ch, LANE), jnp.int32),
            pltpu.VMEM((LANE, H), jnp.float32),
            pltpu.VMEM((LANE, H), jnp.float32),
            pltpu.VMEM((obr, H), jnp.float32),
            pltpu.VMEM_SHARED((Np, H), jnp.float32),
            pltpu.SemaphoreType.DMA,
            pltpu.SemaphoreType.DMA,
        ],
    )


def _prep_body(degp_ref, s2_ref):
    deg_out = jnp.maximum(degp_ref[0] + degp_ref[2], 1.0)
    deg_in = jnp.maximum(degp_ref[1] + degp_ref[3], 1.0)
    s2_ref[...] = jnp.concatenate(
        [lax.rsqrt(deg_out)[None], lax.rsqrt(deg_in)[None]], axis=0)


def _tc_prep(degp):
    return pl.pallas_call(
        _prep_body,
        out_shape=jax.ShapeDtypeStruct((2, Np), jnp.float32),
    )(degp)


def _mm_scale_body(x_ref, w_ref, so_ref, o_ref):
    o_ref[...] = jnp.dot(x_ref[...], w_ref[...],
                         preferred_element_type=jnp.float32) * so_ref[...]


def _tc_mm_scale(x, W1, so_col):
    grid = (N // ROWBLK,)
    return pl.pallas_call(
        _mm_scale_body,
        grid=grid,
        in_specs=[
            pl.BlockSpec((ROWBLK, D), lambda i: (i, 0)),
            pl.BlockSpec((D, H), lambda i: (0, 0)),
            pl.BlockSpec((ROWBLK, 1), lambda i: (i, 0)),
        ],
        out_specs=pl.BlockSpec((ROWBLK, H), lambda i: (i, 0)),
        out_shape=jax.ShapeDtypeStruct((N, H), jnp.float32),
    )(x, W1, so_col)


def _layer_body(p_ref, si_ref, so_ref, b_ref, w_ref, o_ref):
    h = jax.nn.relu((p_ref[0] + p_ref[1]) * si_ref[...] + b_ref[...])
    o_ref[...] = jnp.dot(h * so_ref[...], w_ref[...],
                         preferred_element_type=jnp.float32)


def _tc_layer(aggp, si_col, so_col, b1r, W2):
    grid = (N // ROWBLK,)
    return pl.pallas_call(
        _layer_body,
        grid=grid,
        in_specs=[
            pl.BlockSpec((2, ROWBLK, H), lambda i: (0, i, 0)),
            pl.BlockSpec((ROWBLK, 1), lambda i: (i, 0)),
            pl.BlockSpec((ROWBLK, 1), lambda i: (i, 0)),
            pl.BlockSpec((1, H), lambda i: (0, 0)),
            pl.BlockSpec((H, H), lambda i: (0, 0)),
        ],
        out_specs=pl.BlockSpec((ROWBLK, H), lambda i: (i, 0)),
        out_shape=jax.ShapeDtypeStruct((N, H), jnp.float32),
    )(aggp, si_col, so_col, b1r, W2)


def _final_body(p_ref, si_ref, b_ref, wc_ref, bc_ref, wr_ref, br_ref,
                o1_ref, o2_ref, acc_ref):
    i = pl.program_id(0)
    h = jax.nn.relu((p_ref[0] + p_ref[1]) * si_ref[...] + b_ref[...])
    ps = jnp.sum(h, axis=0, keepdims=True)

    @pl.when(i == 0)
    def _():
        acc_ref[...] = ps

    @pl.when(i > 0)
    def _():
        acc_ref[...] = acc_ref[...] + ps

    @pl.when(i == pl.num_programs(0) - 1)
    def _():
        hg = acc_ref[...] * (1.0 / N)
        o1_ref[...] = jnp.dot(hg, wc_ref[...],
                              preferred_element_type=jnp.float32) + bc_ref[...]
        o2_ref[...] = jnp.dot(hg, wr_ref[...],
                              preferred_element_type=jnp.float32) + br_ref[...]


def _tc_final(aggp, si_col, b2r, Wc, bcr, Wr, brr):
    grid = (N // ROWBLK,)
    nc = Wc.shape[1]
    nr = Wr.shape[1]
    return pl.pallas_call(
        _final_body,
        grid=grid,
        in_specs=[
            pl.BlockSpec((2, ROWBLK, H), lambda i: (0, i, 0)),
            pl.BlockSpec((ROWBLK, 1), lambda i: (i, 0)),
            pl.BlockSpec((1, H), lambda i: (0, 0)),
            pl.BlockSpec((H, nc), lambda i: (0, 0)),
            pl.BlockSpec((1, nc), lambda i: (0, 0)),
            pl.BlockSpec((H, nr), lambda i: (0, 0)),
            pl.BlockSpec((1, nr), lambda i: (0, 0)),
        ],
        out_specs=[
            pl.BlockSpec((1, nc), lambda i: (0, 0)),
            pl.BlockSpec((1, nr), lambda i: (0, 0)),
        ],
        out_shape=[
            jax.ShapeDtypeStruct((1, nc), jnp.float32),
            jax.ShapeDtypeStruct((1, nr), jnp.float32),
        ],
        scratch_shapes=[pltpu.VMEM((1, H), jnp.float32)],
    )(aggp, si_col, b2r, Wc, bcr, Wr, brr)


def kernel(x, edge_index, W1, b1, W2, b2, Wc, bc, Wr, br):
    E = edge_index.shape[1]
    per_w = -(-E // (NW * LANE)) * LANE
    ch = per_w // LANE
    if ch % 2:
        ch += 1
        per_w = ch * LANE
    e_pad = NW * per_w - E

    src = edge_index[0]
    dst = edge_index[1]
    srcg = jnp.concatenate(
        [src, jnp.zeros((e_pad,), jnp.int32)]).reshape(NW * ch, LANE)
    srcd = jnp.concatenate(
        [src, jnp.full((e_pad,), N, jnp.int32)]).reshape(NW * ch, LANE)
    dstg = jnp.concatenate(
        [dst, jnp.full((e_pad,), N, jnp.int32)]).reshape(NW * ch, LANE)

    degp = _make_degrees(ch)(srcd, dstg)
    s2 = _tc_prep(degp.reshape(4, Np))
    so_col = s2[0].reshape(Np, 1)
    si_col = s2[1].reshape(Np, 1)

    segsum = _make_segsum(ch)
    t1 = _tc_mm_scale(x, W1, so_col)
    aggp1 = segsum(t1, srcg, dstg).reshape(2, Np, H)
    t2 = _tc_layer(aggp1, si_col, so_col, b1.reshape(1, H), W2)
    aggp2 = segsum(t2, srcg, dstg).reshape(2, Np, H)
    out_cat, out_cont = _tc_final(
        aggp2, si_col, b2.reshape(1, H), Wc, bc.reshape(1, Wc.shape[1]),
        Wr, br.reshape(1, Wr.shape[1]))
    return (out_cat, out_cont)


# speedup vs baseline: 3.6010x; 3.6010x over previous
import functools

import jax
import jax.numpy as jnp
from jax import lax
from jax.experimental import pallas as pl
from jax.experimental.pallas import tpu as pltpu
from jax.experimental.pallas import tpu_sc as plsc

N = 10000
D = 128
H = 128
NC = 2
NS = 16
NW = NC * NS
LANE = 128
Np = 10240
RPT = Np // NS
ROWBLK = 1000

_mesh = plsc.VectorSubcoreMesh(
    core_axis_name="c", subcore_axis_name="s", num_cores=NC, num_subcores=NS)


def _make_degrees(ch):
    def body(srcd, dstd, degp, src_v, dst_v, ones_v, zbuf, acc_out, acc_in):
        c = lax.axis_index("c")
        s = lax.axis_index("s")
        wid = c * NS + s
        pltpu.sync_copy(srcd.at[pl.ds(wid * ch, ch)], src_v)
        pltpu.sync_copy(dstd.at[pl.ds(wid * ch, ch)], dst_v)
        for i in range(LANE // 16):
            ones_v[pl.ds(i * 16, 16)] = jnp.ones((16,), jnp.float32)
        for i in range(RPT // 16):
            zbuf[pl.ds(i * 16, 16)] = jnp.zeros((16,), jnp.float32)
        pltpu.sync_copy(zbuf, acc_out.at[pl.ds(s * RPT, RPT)])
        pltpu.sync_copy(zbuf, acc_in.at[pl.ds(s * RPT, RPT)])
        plsc.subcore_barrier()

        def step(j, carry):
            pltpu.sync_copy(ones_v, acc_out.at[src_v.at[j]], add=True)
            pltpu.sync_copy(ones_v, acc_in.at[dst_v.at[j]], add=True)
            return carry

        lax.fori_loop(0, ch, step, 0)
        plsc.subcore_barrier()
        pltpu.sync_copy(acc_out.at[pl.ds(s * RPT, RPT)], zbuf)
        pltpu.sync_copy(zbuf, degp.at[pl.ds((2 * c) * Np + s * RPT, RPT)])
        pltpu.sync_copy(acc_in.at[pl.ds(s * RPT, RPT)], zbuf)
        pltpu.sync_copy(zbuf, degp.at[pl.ds((2 * c + 1) * Np + s * RPT, RPT)])

    return pl.kernel(
        body,
        out_type=jax.ShapeDtypeStruct((4 * Np,), jnp.float32),
        mesh=_mesh,
        scratch_types=[
            pltpu.VMEM((ch, LANE), jnp.int32),
            pltpu.VMEM((ch, LANE), jnp.int32),
            pltpu.VMEM((LANE,), jnp.float32),
            pltpu.VMEM((RPT,), jnp.float32),
            pltpu.VMEM_SHARED((Np,), jnp.float32),
            pltpu.VMEM_SHARED((Np,), jnp.float32),
        ],
    )


IB = 16


def _make_segsum(ch):
    assert ch % IB == 0 and IB % 2 == 0
    nblk = ch // IB

    def body(t, srcg, dstg, aggp, src_v, dst_v, buf0, buf1, acc, sem0, sem1):
        c = lax.axis_index("c")
        s = lax.axis_index("s")
        wid = c * NS + s

        def zrow(r, carry):
            for k in range(H // 16):
                buf0[r, pl.ds(k * 16, 16)] = jnp.zeros((16,), jnp.float32)
            return carry

        lax.fori_loop(0, LANE, zrow, 0)
        for hh in range(RPT // LANE):
            pltpu.sync_copy(buf0, acc.at[pl.ds(s * RPT + hh * LANE, LANE)])
        plsc.subcore_barrier()

        def block(blk, carry):
            base = wid * ch + blk * IB
            pltpu.sync_copy(srcg.at[pl.ds(base, IB)], src_v)
            pltpu.sync_copy(dstg.at[pl.ds(base, IB)], dst_v)
            pltpu.async_copy(t.at[src_v.at[0]], buf0, sem0)
            pltpu.async_copy(t.at[src_v.at[1]], buf1, sem1)

            def step(i, carry2):
                j = 2 * i
                pltpu.make_async_copy(t.at[src_v.at[j]], buf0, sem0).wait()
                pltpu.sync_copy(buf0, acc.at[dst_v.at[j]], add=True)
                pltpu.async_copy(t.at[src_v.at[j + 2]], buf0, sem0)
                pltpu.make_async_copy(
                    t.at[src_v.at[j + 1]], buf1, sem1).wait()
                pltpu.sync_copy(buf1, acc.at[dst_v.at[j + 1]], add=True)
                pltpu.async_copy(t.at[src_v.at[j + 3]], buf1, sem1)
                return carry2

            lax.fori_loop(0, IB // 2 - 1, step, 0)
            jlast = IB - 2
            pltpu.make_async_copy(t.at[src_v.at[jlast]], buf0, sem0).wait()
            pltpu.sync_copy(buf0, acc.at[dst_v.at[jlast]], add=True)
            pltpu.make_async_copy(t.at[src_v.at[jlast + 1]], buf1, sem1).wait()
            pltpu.sync_copy(buf1, acc.at[dst_v.at[jlast + 1]], add=True)
            return carry

        lax.fori_loop(0, nblk, block, 0)

        plsc.subcore_barrier()
        for hh in range(RPT // LANE):
            pltpu.sync_copy(acc.at[pl.ds(s * RPT + hh * LANE, LANE)], buf0)
            pltpu.sync_copy(
                buf0, aggp.at[pl.ds(c * Np + s * RPT + hh * LANE, LANE)])

    return pl.kernel(
        body,
        out_type=jax.ShapeDtypeStruct((2 * Np, H), jnp.float32),
        mesh=_mesh,
        scratch_types=[
            pltpu.VMEM((IB, LANE), jnp.int32),
            pltpu.VMEM((IB, LANE), jnp.int32),
            pltpu.VMEM((LANE, H), jnp.float32),
            pltpu.VMEM((LANE, H), jnp.float32),
            pltpu.VMEM_SHARED((Np, H), jnp.float32),
            pltpu.SemaphoreType.DMA,
            pltpu.SemaphoreType.DMA,
        ],
    )


def _prep_body(degp_ref, s2_ref):
    deg_out = jnp.maximum(degp_ref[0] + degp_ref[2], 1.0)
    deg_in = jnp.maximum(degp_ref[1] + degp_ref[3], 1.0)
    s2_ref[...] = jnp.concatenate(
        [lax.rsqrt(deg_out)[None], lax.rsqrt(deg_in)[None]], axis=0)


def _tc_prep(degp):
    return pl.pallas_call(
        _prep_body,
        out_shape=jax.ShapeDtypeStruct((2, Np), jnp.float32),
    )(degp)


def _mm_scale_body(x_ref, w_ref, so_ref, o_ref):
    o_ref[...] = jnp.dot(x_ref[...], w_ref[...],
                         preferred_element_type=jnp.float32) * so_ref[...]


def _tc_mm_scale(x, W1, so_col):
    grid = (N // ROWBLK,)
    return pl.pallas_call(
        _mm_scale_body,
        grid=grid,
        in_specs=[
            pl.BlockSpec((ROWBLK, D), lambda i: (i, 0)),
            pl.BlockSpec((D, H), lambda i: (0, 0)),
            pl.BlockSpec((ROWBLK, 1), lambda i: (i, 0)),
        ],
        out_specs=pl.BlockSpec((ROWBLK, H), lambda i: (i, 0)),
        out_shape=jax.ShapeDtypeStruct((N, H), jnp.float32),
    )(x, W1, so_col)


def _layer_body(p_ref, si_ref, so_ref, b_ref, w_ref, o_ref):
    h = jax.nn.relu((p_ref[0] + p_ref[1]) * si_ref[...] + b_ref[...])
    o_ref[...] = jnp.dot(h * so_ref[...], w_ref[...],
                         preferred_element_type=jnp.float32)


def _tc_layer(aggp, si_col, so_col, b1r, W2):
    grid = (N // ROWBLK,)
    return pl.pallas_call(
        _layer_body,
        grid=grid,
        in_specs=[
            pl.BlockSpec((2, ROWBLK, H), lambda i: (0, i, 0)),
            pl.BlockSpec((ROWBLK, 1), lambda i: (i, 0)),
            pl.BlockSpec((ROWBLK, 1), lambda i: (i, 0)),
            pl.BlockSpec((1, H), lambda i: (0, 0)),
            pl.BlockSpec((H, H), lambda i: (0, 0)),
        ],
        out_specs=pl.BlockSpec((ROWBLK, H), lambda i: (i, 0)),
        out_shape=jax.ShapeDtypeStruct((N, H), jnp.float32),
    )(aggp, si_col, so_col, b1r, W2)


def _final_body(p_ref, si_ref, b_ref, wc_ref, bc_ref, wr_ref, br_ref,
                o1_ref, o2_ref, acc_ref):
    i = pl.program_id(0)
    h = jax.nn.relu((p_ref[0] + p_ref[1]) * si_ref[...] + b_ref[...])
    ps = jnp.sum(h, axis=0, keepdims=True)

    @pl.when(i == 0)
    def _():
        acc_ref[...] = ps

    @pl.when(i > 0)
    def _():
        acc_ref[...] = acc_ref[...] + ps

    @pl.when(i == pl.num_programs(0) - 1)
    def _():
        hg = acc_ref[...] * (1.0 / N)
        o1_ref[...] = jnp.dot(hg, wc_ref[...],
                              preferred_element_type=jnp.float32) + bc_ref[...]
        o2_ref[...] = jnp.dot(hg, wr_ref[...],
                              preferred_element_type=jnp.float32) + br_ref[...]


def _tc_final(aggp, si_col, b2r, Wc, bcr, Wr, brr):
    grid = (N // ROWBLK,)
    nc = Wc.shape[1]
    nr = Wr.shape[1]
    return pl.pallas_call(
        _final_body,
        grid=grid,
        in_specs=[
            pl.BlockSpec((2, ROWBLK, H), lambda i: (0, i, 0)),
            pl.BlockSpec((ROWBLK, 1), lambda i: (i, 0)),
            pl.BlockSpec((1, H), lambda i: (0, 0)),
            pl.BlockSpec((H, nc), lambda i: (0, 0)),
            pl.BlockSpec((1, nc), lambda i: (0, 0)),
            pl.BlockSpec((H, nr), lambda i: (0, 0)),
            pl.BlockSpec((1, nr), lambda i: (0, 0)),
        ],
        out_specs=[
            pl.BlockSpec((1, nc), lambda i: (0, 0)),
            pl.BlockSpec((1, nr), lambda i: (0, 0)),
        ],
        out_shape=[
            jax.ShapeDtypeStruct((1, nc), jnp.float32),
            jax.ShapeDtypeStruct((1, nr), jnp.float32),
        ],
        scratch_shapes=[pltpu.VMEM((1, H), jnp.float32)],
    )(aggp, si_col, b2r, Wc, bcr, Wr, brr)


def kernel(x, edge_index, W1, b1, W2, b2, Wc, bc, Wr, br):
    E = edge_index.shape[1]
    per_w = -(-E // (NW * LANE)) * LANE
    ch = per_w // LANE
    if ch % 2:
        ch += 1
        per_w = ch * LANE
    e_pad = NW * per_w - E

    src = edge_index[0]
    dst = edge_index[1]
    srcg = jnp.concatenate(
        [src, jnp.zeros((e_pad,), jnp.int32)]).reshape(NW * ch, LANE)
    srcd = jnp.concatenate(
        [src, jnp.full((e_pad,), N, jnp.int32)]).reshape(NW * ch, LANE)
    dstg = jnp.concatenate(
        [dst, jnp.full((e_pad,), N, jnp.int32)]).reshape(NW * ch, LANE)

    degp = _make_degrees(ch)(srcd, dstg)
    s2 = _tc_prep(degp.reshape(4, Np))
    so_col = s2[0].reshape(Np, 1)
    si_col = s2[1].reshape(Np, 1)

    segsum = _make_segsum(ch)
    t1 = _tc_mm_scale(x, W1, so_col)
    aggp1 = segsum(t1, srcg, dstg).reshape(2, Np, H)
    t2 = _tc_layer(aggp1, si_col, so_col, b1.reshape(1, H), W2)
    aggp2 = segsum(t2, srcg, dstg).reshape(2, Np, H)
    out_cat, out_cont = _tc_final(
        aggp2, si_col, b2.reshape(1, H), Wc, bc.reshape(1, Wc.shape[1]),
        Wr, br.reshape(1, Wr.shape[1]))
    return (out_cat, out_cont)


# speedup vs baseline: 11.2826x; 3.1332x over previous
import functools

import jax
import jax.numpy as jnp
from jax import lax
from jax.experimental import pallas as pl
from jax.experimental.pallas import tpu as pltpu
from jax.experimental.pallas import tpu_sc as plsc

N = 10000
D = 128
H = 128
NC = 2
NS = 16
NW = NC * NS
LANE = 128
Np = 10240
RPT = Np // NS
ROWBLK = 1000

_mesh = plsc.VectorSubcoreMesh(
    core_axis_name="c", subcore_axis_name="s", num_cores=NC, num_subcores=NS)


def _make_degrees(ch):
    def body(srcd, dstd, degp, src_v, dst_v, ones_v, zbuf, acc_out, acc_in):
        c = lax.axis_index("c")
        s = lax.axis_index("s")
        wid = c * NS + s
        pltpu.sync_copy(srcd.at[pl.ds(wid * ch, ch)], src_v)
        pltpu.sync_copy(dstd.at[pl.ds(wid * ch, ch)], dst_v)
        for i in range(LANE // 16):
            ones_v[pl.ds(i * 16, 16)] = jnp.ones((16,), jnp.float32)
        for i in range(RPT // 16):
            zbuf[pl.ds(i * 16, 16)] = jnp.zeros((16,), jnp.float32)
        pltpu.sync_copy(zbuf, acc_out.at[pl.ds(s * RPT, RPT)])
        pltpu.sync_copy(zbuf, acc_in.at[pl.ds(s * RPT, RPT)])
        plsc.subcore_barrier()

        def step(j, carry):
            pltpu.sync_copy(ones_v, acc_out.at[src_v.at[j]], add=True)
            pltpu.sync_copy(ones_v, acc_in.at[dst_v.at[j]], add=True)
            return carry

        lax.fori_loop(0, ch, step, 0)
        plsc.subcore_barrier()
        pltpu.sync_copy(acc_out.at[pl.ds(s * RPT, RPT)], zbuf)
        pltpu.sync_copy(zbuf, degp.at[pl.ds((2 * c) * Np + s * RPT, RPT)])
        pltpu.sync_copy(acc_in.at[pl.ds(s * RPT, RPT)], zbuf)
        pltpu.sync_copy(zbuf, degp.at[pl.ds((2 * c + 1) * Np + s * RPT, RPT)])

    return pl.kernel(
        body,
        out_type=jax.ShapeDtypeStruct((4 * Np,), jnp.float32),
        mesh=_mesh,
        scratch_types=[
            pltpu.VMEM((ch, LANE), jnp.int32),
            pltpu.VMEM((ch, LANE), jnp.int32),
            pltpu.VMEM((LANE,), jnp.float32),
            pltpu.VMEM((RPT,), jnp.float32),
            pltpu.VMEM_SHARED((Np,), jnp.float32),
            pltpu.VMEM_SHARED((Np,), jnp.float32),
        ],
    )


IB = 16


def _make_segsum(ch):
    assert ch % IB == 0 and IB % 2 == 0
    nblk = ch // IB

    def body(t, srcg, dstg, aggp, src_v, dst_v, buf0, buf1, acc, sem0, sem1):
        c = lax.axis_index("c")
        s = lax.axis_index("s")
        wid = c * NS + s

        def zrow(r, carry):
            for k in range(H // 16):
                buf0[r, pl.ds(k * 16, 16)] = jnp.zeros((16,), jnp.float32)
            return carry

        lax.fori_loop(0, LANE, zrow, 0)
        for hh in range(RPT // LANE):
            pltpu.sync_copy(buf0, acc.at[pl.ds(s * RPT + hh * LANE, LANE)])
        plsc.subcore_barrier()

        def block(blk, carry):
            base = wid * ch + blk * IB
            pltpu.sync_copy(srcg.at[pl.ds(base, IB)], src_v)
            pltpu.sync_copy(dstg.at[pl.ds(base, IB)], dst_v)
            pltpu.async_copy(t.at[src_v.at[0]], buf0, sem0)
            pltpu.async_copy(t.at[src_v.at[1]], buf1, sem1)

            def step(i, carry2):
                j = 2 * i
                pltpu.make_async_copy(t.at[src_v.at[j]], buf0, sem0).wait()
                pltpu.sync_copy(buf0, acc.at[dst_v.at[j]], add=True)
                pltpu.async_copy(t.at[src_v.at[j + 2]], buf0, sem0)
                pltpu.make_async_copy(
                    t.at[src_v.at[j + 1]], buf1, sem1).wait()
                pltpu.sync_copy(buf1, acc.at[dst_v.at[j + 1]], add=True)
                pltpu.async_copy(t.at[src_v.at[j + 3]], buf1, sem1)
                return carry2

            lax.fori_loop(0, IB // 2 - 1, step, 0)
            jlast = IB - 2
            pltpu.make_async_copy(t.at[src_v.at[jlast]], buf0, sem0).wait()
            pltpu.sync_copy(buf0, acc.at[dst_v.at[jlast]], add=True)
            pltpu.make_async_copy(t.at[src_v.at[jlast + 1]], buf1, sem1).wait()
            pltpu.sync_copy(buf1, acc.at[dst_v.at[jlast + 1]], add=True)
            return carry

        lax.fori_loop(0, nblk, block, 0)

        plsc.subcore_barrier()
        for hh in range(RPT // LANE):
            pltpu.sync_copy(acc.at[pl.ds(s * RPT + hh * LANE, LANE)], buf0)
            pltpu.sync_copy(
                buf0, aggp.at[pl.ds(c * Np + s * RPT + hh * LANE, LANE)])

    return pl.kernel(
        body,
        out_type=jax.ShapeDtypeStruct((2 * Np, H), jnp.float32),
        mesh=_mesh,
        scratch_types=[
            pltpu.VMEM((IB, LANE), jnp.int32),
            pltpu.VMEM((IB, LANE), jnp.int32),
            pltpu.VMEM((LANE, H), jnp.float32),
            pltpu.VMEM((LANE, H), jnp.float32),
            pltpu.VMEM_SHARED((Np, H), jnp.float32),
            pltpu.SemaphoreType.DMA,
            pltpu.SemaphoreType.DMA,
        ],
    )


def _prep_body(degp_ref, s2_ref):
    deg_out = jnp.maximum(degp_ref[0] + degp_ref[2], 1.0)
    deg_in = jnp.maximum(degp_ref[1] + degp_ref[3], 1.0)
    s2_ref[...] = jnp.concatenate(
        [lax.rsqrt(deg_out)[None], lax.rsqrt(deg_in)[None]], axis=0)


def _tc_prep(degp):
    return pl.pallas_call(
        _prep_body,
        out_shape=jax.ShapeDtypeStruct((2, Np), jnp.float32),
    )(degp)


def _mm_scale_body(x_ref, w_ref, so_ref, o_ref):
    o_ref[...] = jnp.dot(x_ref[...], w_ref[...],
                         preferred_element_type=jnp.float32) * so_ref[...]


def _tc_mm_scale(x, W1, so_col):
    grid = (N // ROWBLK,)
    return pl.pallas_call(
        _mm_scale_body,
        grid=grid,
        in_specs=[
            pl.BlockSpec((ROWBLK, D), lambda i: (i, 0)),
            pl.BlockSpec((D, H), lambda i: (0, 0)),
            pl.BlockSpec((ROWBLK, 1), lambda i: (i, 0)),
        ],
        out_specs=pl.BlockSpec((ROWBLK, H), lambda i: (i, 0)),
        out_shape=jax.ShapeDtypeStruct((N, H), jnp.float32),
    )(x, W1, so_col)


def _layer_body(p_ref, si_ref, so_ref, b_ref, w_ref, o_ref):
    h = jax.nn.relu((p_ref[0] + p_ref[1]) * si_ref[...] + b_ref[...])
    o_ref[...] = jnp.dot(h * so_ref[...], w_ref[...],
                         preferred_element_type=jnp.float32)


def _tc_layer(aggp, si_col, so_col, b1r, W2):
    grid = (N // ROWBLK,)
    return pl.pallas_call(
        _layer_body,
        grid=grid,
        in_specs=[
            pl.BlockSpec((2, ROWBLK, H), lambda i: (0, i, 0)),
            pl.BlockSpec((ROWBLK, 1), lambda i: (i, 0)),
            pl.BlockSpec((ROWBLK, 1), lambda i: (i, 0)),
            pl.BlockSpec((1, H), lambda i: (0, 0)),
            pl.BlockSpec((H, H), lambda i: (0, 0)),
        ],
        out_specs=pl.BlockSpec((ROWBLK, H), lambda i: (i, 0)),
        out_shape=jax.ShapeDtypeStruct((N, H), jnp.float32),
    )(aggp, si_col, so_col, b1r, W2)


def _final_body(p_ref, si_ref, b_ref, wc_ref, bc_ref, wr_ref, br_ref,
                o1_ref, o2_ref, acc_ref):
    i = pl.program_id(0)
    h = jax.nn.relu((p_ref[0] + p_ref[1]) * si_ref[...] + b_ref[...])
    ps = jnp.sum(h, axis=0, keepdims=True)

    @pl.when(i == 0)
    def _():
        acc_ref[...] = ps

    @pl.when(i > 0)
    def _():
        acc_ref[...] = acc_ref[...] + ps

    @pl.when(i == pl.num_programs(0) - 1)
    def _():
        hg = acc_ref[...] * (1.0 / N)
        o1_ref[...] = jnp.dot(hg, wc_ref[...],
                              preferred_element_type=jnp.float32) + bc_ref[...]
        o2_ref[...] = jnp.dot(hg, wr_ref[...],
                              preferred_element_type=jnp.float32) + br_ref[...]


def _tc_final(aggp, si_col, b2r, Wc, bcr, Wr, brr):
    grid = (N // ROWBLK,)
    nc = Wc.shape[1]
    nr = Wr.shape[1]
    return pl.pallas_call(
        _final_body,
        grid=grid,
        in_specs=[
            pl.BlockSpec((2, ROWBLK, H), lambda i: (0, i, 0)),
            pl.BlockSpec((ROWBLK, 1), lambda i: (i, 0)),
            pl.BlockSpec((1, H), lambda i: (0, 0)),
            pl.BlockSpec((H, nc), lambda i: (0, 0)),
            pl.BlockSpec((1, nc), lambda i: (0, 0)),
            pl.BlockSpec((H, nr), lambda i: (0, 0)),
            pl.BlockSpec((1, nr), lambda i: (0, 0)),
        ],
        out_specs=[
            pl.BlockSpec((1, nc), lambda i: (0, 0)),
            pl.BlockSpec((1, nr), lambda i: (0, 0)),
        ],
        out_shape=[
            jax.ShapeDtypeStruct((1, nc), jnp.float32),
            jax.ShapeDtypeStruct((1, nr), jnp.float32),
        ],
        scratch_shapes=[pltpu.VMEM((1, H), jnp.float32)],
    )(aggp, si_col, b2r, Wc, bcr, Wr, brr)


def kernel(x, edge_index, W1, b1, W2, b2, Wc, bc, Wr, br):
    E = edge_index.shape[1]
    per_w = -(-E // (NW * LANE)) * LANE
    ch = per_w // LANE
    if ch % 2:
        ch += 1
        per_w = ch * LANE
    e_pad = NW * per_w - E

    src = edge_index[0]
    dst = edge_index[1]
    pad_i = jnp.arange(e_pad, dtype=jnp.int32)
    pad_node = N + pad_i % (Np - N)
    srcg = jnp.concatenate([src, pad_i % N]).reshape(NW * ch, LANE)
    srcd = jnp.concatenate([src, pad_node]).reshape(NW * ch, LANE)
    dstg = jnp.concatenate([dst, pad_node]).reshape(NW * ch, LANE)

    degp = _make_degrees(ch)(srcd, dstg)
    s2 = _tc_prep(degp.reshape(4, Np))
    so_col = s2[0].reshape(Np, 1)
    si_col = s2[1].reshape(Np, 1)

    segsum = _make_segsum(ch)
    t1 = _tc_mm_scale(x, W1, so_col)
    aggp1 = segsum(t1, srcg, dstg).reshape(2, Np, H)
    t2 = _tc_layer(aggp1, si_col, so_col, b1.reshape(1, H), W2)
    aggp2 = segsum(t2, srcg, dstg).reshape(2, Np, H)
    out_cat, out_cont = _tc_final(
        aggp2, si_col, b2.reshape(1, H), Wc, bc.reshape(1, Wc.shape[1]),
        Wr, br.reshape(1, Wr.shape[1]))
    return (out_cat, out_cont)


# speedup vs baseline: 11.8662x; 1.0517x over previous
import functools

import jax
import jax.numpy as jnp
from jax import lax
from jax.experimental import pallas as pl
from jax.experimental.pallas import tpu as pltpu
from jax.experimental.pallas import tpu_sc as plsc

N = 10000
D = 128
H = 128
NC = 2
NS = 16
NW = NC * NS
LANE = 128
Np = 10240
RPT = Np // NS
ROWBLK = 1000

_mesh = plsc.VectorSubcoreMesh(
    core_axis_name="c", subcore_axis_name="s", num_cores=NC, num_subcores=NS)


def _make_degrees(ch):
    def body(srcd, dstd, degp, src_v, dst_v, ones_v, zbuf, acc_out, acc_in):
        c = lax.axis_index("c")
        s = lax.axis_index("s")
        wid = c * NS + s
        pltpu.sync_copy(srcd.at[pl.ds(wid * ch, ch)], src_v)
        pltpu.sync_copy(dstd.at[pl.ds(wid * ch, ch)], dst_v)
        for i in range(LANE // 16):
            ones_v[pl.ds(i * 16, 16)] = jnp.ones((16,), jnp.float32)
        for i in range(RPT // 16):
            zbuf[pl.ds(i * 16, 16)] = jnp.zeros((16,), jnp.float32)
        pltpu.sync_copy(zbuf, acc_out.at[pl.ds(s * RPT, RPT)])
        pltpu.sync_copy(zbuf, acc_in.at[pl.ds(s * RPT, RPT)])
        plsc.subcore_barrier()

        def step(j, carry):
            pltpu.sync_copy(ones_v, acc_out.at[src_v.at[j]], add=True)
            pltpu.sync_copy(ones_v, acc_in.at[dst_v.at[j]], add=True)
            return carry

        lax.fori_loop(0, ch, step, 0)
        plsc.subcore_barrier()
        pltpu.sync_copy(acc_out.at[pl.ds(s * RPT, RPT)], zbuf)
        pltpu.sync_copy(zbuf, degp.at[pl.ds((2 * c) * Np + s * RPT, RPT)])
        pltpu.sync_copy(acc_in.at[pl.ds(s * RPT, RPT)], zbuf)
        pltpu.sync_copy(zbuf, degp.at[pl.ds((2 * c + 1) * Np + s * RPT, RPT)])

    return pl.kernel(
        body,
        out_type=jax.ShapeDtypeStruct((4 * Np,), jnp.float32),
        mesh=_mesh,
        scratch_types=[
            pltpu.VMEM((ch, LANE), jnp.int32),
            pltpu.VMEM((ch, LANE), jnp.int32),
            pltpu.VMEM((LANE,), jnp.float32),
            pltpu.VMEM((RPT,), jnp.float32),
            pltpu.VMEM_SHARED((Np,), jnp.float32),
            pltpu.VMEM_SHARED((Np,), jnp.float32),
        ],
    )


IB = 40


def _make_segsum(ch):
    assert ch % IB == 0 and IB % 2 == 0
    nblk = ch // IB

    def body(t, srcg, dstg, aggp, src_v, dst_v, buf0, buf1, acc, sem0, sem1):
        c = lax.axis_index("c")
        s = lax.axis_index("s")
        wid = c * NS + s

        def zrow(r, carry):
            for k in range(H // 16):
                buf0[r, pl.ds(k * 16, 16)] = jnp.zeros((16,), jnp.float32)
            return carry

        lax.fori_loop(0, LANE, zrow, 0)
        for hh in range(RPT // LANE):
            pltpu.sync_copy(buf0, acc.at[pl.ds(s * RPT + hh * LANE, LANE)])
        plsc.subcore_barrier()

        def block(blk, carry):
            base = wid * ch + blk * IB
            pltpu.sync_copy(srcg.at[pl.ds(base, IB)], src_v)
            pltpu.sync_copy(dstg.at[pl.ds(base, IB)], dst_v)
            pltpu.async_copy(t.at[src_v.at[0]], buf0, sem0)
            pltpu.async_copy(t.at[src_v.at[1]], buf1, sem1)

            def step(i, carry2):
                j = 2 * i
                pltpu.make_async_copy(t.at[src_v.at[j]], buf0, sem0).wait()
                pltpu.sync_copy(buf0, acc.at[dst_v.at[j]], add=True)
                pltpu.async_copy(t.at[src_v.at[j + 2]], buf0, sem0)
                pltpu.make_async_copy(
                    t.at[src_v.at[j + 1]], buf1, sem1).wait()
                pltpu.sync_copy(buf1, acc.at[dst_v.at[j + 1]], add=True)
                pltpu.async_copy(t.at[src_v.at[j + 3]], buf1, sem1)
                return carry2

            lax.fori_loop(0, IB // 2 - 1, step, 0)
            jlast = IB - 2
            pltpu.make_async_copy(t.at[src_v.at[jlast]], buf0, sem0).wait()
            pltpu.sync_copy(buf0, acc.at[dst_v.at[jlast]], add=True)
            pltpu.make_async_copy(t.at[src_v.at[jlast + 1]], buf1, sem1).wait()
            pltpu.sync_copy(buf1, acc.at[dst_v.at[jlast + 1]], add=True)
            return carry

        lax.fori_loop(0, nblk, block, 0)

        plsc.subcore_barrier()
        for hh in range(RPT // LANE):
            pltpu.sync_copy(acc.at[pl.ds(s * RPT + hh * LANE, LANE)], buf0)
            pltpu.sync_copy(
                buf0, aggp.at[pl.ds(c * Np + s * RPT + hh * LANE, LANE)])

    return pl.kernel(
        body,
        out_type=jax.ShapeDtypeStruct((2 * Np, H), jnp.float32),
        mesh=_mesh,
        scratch_types=[
            pltpu.VMEM((IB, LANE), jnp.int32),
            pltpu.VMEM((IB, LANE), jnp.int32),
            pltpu.VMEM((LANE, H), jnp.float32),
            pltpu.VMEM((LANE, H), jnp.float32),
            pltpu.VMEM_SHARED((Np, H), jnp.float32),
            pltpu.SemaphoreType.DMA,
            pltpu.SemaphoreType.DMA,
        ],
    )


def _prep_body(degp_ref, s2_ref):
    deg_out = jnp.maximum(degp_ref[0] + degp_ref[2], 1.0)
    deg_in = jnp.maximum(degp_ref[1] + degp_ref[3], 1.0)
    s2_ref[...] = jnp.concatenate(
        [lax.rsqrt(deg_out)[None], lax.rsqrt(deg_in)[None]], axis=0)


def _tc_prep(degp):
    return pl.pallas_call(
        _prep_body,
        out_shape=jax.ShapeDtypeStruct((2, Np), jnp.float32),
    )(degp)


def _mm_scale_body(x_ref, w_ref, so_ref, o_ref):
    o_ref[...] = jnp.dot(x_ref[...], w_ref[...],
                         preferred_element_type=jnp.float32) * so_ref[...]


def _tc_mm_scale(x, W1, so_col):
    grid = (N // ROWBLK,)
    return pl.pallas_call(
        _mm_scale_body,
        grid=grid,
        in_specs=[
            pl.BlockSpec((ROWBLK, D), lambda i: (i, 0)),
            pl.BlockSpec((D, H), lambda i: (0, 0)),
            pl.BlockSpec((ROWBLK, 1), lambda i: (i, 0)),
        ],
        out_specs=pl.BlockSpec((ROWBLK, H), lambda i: (i, 0)),
        out_shape=jax.ShapeDtypeStruct((N, H), jnp.float32),
    )(x, W1, so_col)


def _layer_body(p_ref, si_ref, so_ref, b_ref, w_ref, o_ref):
    h = jax.nn.relu((p_ref[0] + p_ref[1]) * si_ref[...] + b_ref[...])
    o_ref[...] = jnp.dot(h * so_ref[...], w_ref[...],
                         preferred_element_type=jnp.float32)


def _tc_layer(aggp, si_col, so_col, b1r, W2):
    grid = (N // ROWBLK,)
    return pl.pallas_call(
        _layer_body,
        grid=grid,
        in_specs=[
            pl.BlockSpec((2, ROWBLK, H), lambda i: (0, i, 0)),
            pl.BlockSpec((ROWBLK, 1), lambda i: (i, 0)),
            pl.BlockSpec((ROWBLK, 1), lambda i: (i, 0)),
            pl.BlockSpec((1, H), lambda i: (0, 0)),
            pl.BlockSpec((H, H), lambda i: (0, 0)),
        ],
        out_specs=pl.BlockSpec((ROWBLK, H), lambda i: (i, 0)),
        out_shape=jax.ShapeDtypeStruct((N, H), jnp.float32),
    )(aggp, si_col, so_col, b1r, W2)


def _final_body(p_ref, si_ref, b_ref, wc_ref, bc_ref, wr_ref, br_ref,
                o1_ref, o2_ref, acc_ref):
    i = pl.program_id(0)
    h = jax.nn.relu((p_ref[0] + p_ref[1]) * si_ref[...] + b_ref[...])
    ps = jnp.sum(h, axis=0, keepdims=True)

    @pl.when(i == 0)
    def _():
        acc_ref[...] = ps

    @pl.when(i > 0)
    def _():
        acc_ref[...] = acc_ref[...] + ps

    @pl.when(i == pl.num_programs(0) - 1)
    def _():
        hg = acc_ref[...] * (1.0 / N)
        o1_ref[...] = jnp.dot(hg, wc_ref[...],
                              preferred_element_type=jnp.float32) + bc_ref[...]
        o2_ref[...] = jnp.dot(hg, wr_ref[...],
                              preferred_element_type=jnp.float32) + br_ref[...]


def _tc_final(aggp, si_col, b2r, Wc, bcr, Wr, brr):
    grid = (N // ROWBLK,)
    nc = Wc.shape[1]
    nr = Wr.shape[1]
    return pl.pallas_call(
        _final_body,
        grid=grid,
        in_specs=[
            pl.BlockSpec((2, ROWBLK, H), lambda i: (0, i, 0)),
            pl.BlockSpec((ROWBLK, 1), lambda i: (i, 0)),
            pl.BlockSpec((1, H), lambda i: (0, 0)),
            pl.BlockSpec((H, nc), lambda i: (0, 0)),
            pl.BlockSpec((1, nc), lambda i: (0, 0)),
            pl.BlockSpec((H, nr), lambda i: (0, 0)),
            pl.BlockSpec((1, nr), lambda i: (0, 0)),
        ],
        out_specs=[
            pl.BlockSpec((1, nc), lambda i: (0, 0)),
            pl.BlockSpec((1, nr), lambda i: (0, 0)),
        ],
        out_shape=[
            jax.ShapeDtypeStruct((1, nc), jnp.float32),
            jax.ShapeDtypeStruct((1, nr), jnp.float32),
        ],
        scratch_shapes=[pltpu.VMEM((1, H), jnp.float32)],
    )(aggp, si_col, b2r, Wc, bcr, Wr, brr)


def kernel(x, edge_index, W1, b1, W2, b2, Wc, bc, Wr, br):
    E = edge_index.shape[1]
    per_w = -(-E // (NW * LANE)) * LANE
    ch = per_w // LANE
    if ch % 2:
        ch += 1
        per_w = ch * LANE
    e_pad = NW * per_w - E

    src = edge_index[0]
    dst = edge_index[1]
    pad_i = jnp.arange(e_pad, dtype=jnp.int32)
    pad_node = N + pad_i % (Np - N)
    srcg = jnp.concatenate([src, pad_i % N]).reshape(NW * ch, LANE)
    srcd = jnp.concatenate([src, pad_node]).reshape(NW * ch, LANE)
    dstg = jnp.concatenate([dst, pad_node]).reshape(NW * ch, LANE)

    degp = _make_degrees(ch)(srcd, dstg)
    s2 = _tc_prep(degp.reshape(4, Np))
    so_col = s2[0].reshape(Np, 1)
    si_col = s2[1].reshape(Np, 1)

    segsum = _make_segsum(ch)
    t1 = _tc_mm_scale(x, W1, so_col)
    aggp1 = segsum(t1, srcg, dstg).reshape(2, Np, H)
    t2 = _tc_layer(aggp1, si_col, so_col, b1.reshape(1, H), W2)
    aggp2 = segsum(t2, srcg, dstg).reshape(2, Np, H)
    out_cat, out_cont = _tc_final(
        aggp2, si_col, b2.reshape(1, H), Wc, bc.reshape(1, Wc.shape[1]),
        Wr, br.reshape(1, Wr.shape[1]))
    return (out_cat, out_cont)


# speedup vs baseline: 11.9113x; 1.0038x over previous
import functools

import jax
import jax.numpy as jnp
from jax import lax
from jax.experimental import pallas as pl
from jax.experimental.pallas import tpu as pltpu
from jax.experimental.pallas import tpu_sc as plsc

N = 10000
D = 128
H = 128
NC = 2
NS = 16
NW = NC * NS
LANE = 128
Np = 10240
RPT = Np // NS
ROWBLK = 1000

_mesh = plsc.VectorSubcoreMesh(
    core_axis_name="c", subcore_axis_name="s", num_cores=NC, num_subcores=NS)


def _make_degrees(ch, n_edges):
    def body(srcd, dstd, degp, src_v, dst_v, ones_v, zbuf, acc_out, acc_in):
        c = lax.axis_index("c")
        s = lax.axis_index("s")
        wid = c * NS + s
        pltpu.sync_copy(srcd.at[pl.ds(wid * ch, ch)], src_v)
        pltpu.sync_copy(dstd.at[pl.ds(wid * ch, ch)], dst_v)
        for i in range(LANE // 16):
            ones_v[pl.ds(i * 16, 16)] = jnp.ones((16,), jnp.float32)
        for i in range(RPT // 16):
            zbuf[pl.ds(i * 16, 16)] = jnp.zeros((16,), jnp.float32)
        pltpu.sync_copy(zbuf, acc_out.at[pl.ds(s * RPT, RPT)])
        pltpu.sync_copy(zbuf, acc_in.at[pl.ds(s * RPT, RPT)])
        plsc.subcore_barrier()

        nch = jnp.clip((n_edges - wid * ch * LANE + LANE - 1) // LANE, 0, ch)

        def step(j, carry):
            pltpu.sync_copy(ones_v, acc_out.at[src_v.at[j]], add=True)
            pltpu.sync_copy(ones_v, acc_in.at[dst_v.at[j]], add=True)
            return carry

        lax.fori_loop(0, nch, step, 0)
        plsc.subcore_barrier()
        pltpu.sync_copy(acc_out.at[pl.ds(s * RPT, RPT)], zbuf)
        pltpu.sync_copy(zbuf, degp.at[pl.ds((2 * c) * Np + s * RPT, RPT)])
        pltpu.sync_copy(acc_in.at[pl.ds(s * RPT, RPT)], zbuf)
        pltpu.sync_copy(zbuf, degp.at[pl.ds((2 * c + 1) * Np + s * RPT, RPT)])

    return pl.kernel(
        body,
        out_type=jax.ShapeDtypeStruct((4 * Np,), jnp.float32),
        mesh=_mesh,
        scratch_types=[
            pltpu.VMEM((ch, LANE), jnp.int32),
            pltpu.VMEM((ch, LANE), jnp.int32),
            pltpu.VMEM((LANE,), jnp.float32),
            pltpu.VMEM((RPT,), jnp.float32),
            pltpu.VMEM_SHARED((Np,), jnp.float32),
            pltpu.VMEM_SHARED((Np,), jnp.float32),
        ],
    )


IB = 40


def _make_segsum(ch):
    assert ch % IB == 0 and IB % 2 == 0
    nblk = ch // IB

    def body(t, srcg, dstg, aggp, src_v, dst_v, buf0, buf1, acc, sem0, sem1):
        c = lax.axis_index("c")
        s = lax.axis_index("s")
        wid = c * NS + s

        def zrow(r, carry):
            for k in range(H // 16):
                buf0[r, pl.ds(k * 16, 16)] = jnp.zeros((16,), jnp.float32)
            return carry

        lax.fori_loop(0, LANE, zrow, 0)
        for hh in range(RPT // LANE):
            pltpu.sync_copy(buf0, acc.at[pl.ds(s * RPT + hh * LANE, LANE)])
        plsc.subcore_barrier()

        def block(blk, carry):
            base = wid * ch + blk * IB
            pltpu.sync_copy(srcg.at[pl.ds(base, IB)], src_v)
            pltpu.sync_copy(dstg.at[pl.ds(base, IB)], dst_v)
            pltpu.async_copy(t.at[src_v.at[0]], buf0, sem0)
            pltpu.async_copy(t.at[src_v.at[1]], buf1, sem1)

            def step(i, carry2):
                j = 2 * i
                pltpu.make_async_copy(t.at[src_v.at[j]], buf0, sem0).wait()
                pltpu.sync_copy(buf0, acc.at[dst_v.at[j]], add=True)
                pltpu.async_copy(t.at[src_v.at[j + 2]], buf0, sem0)
                pltpu.make_async_copy(
                    t.at[src_v.at[j + 1]], buf1, sem1).wait()
                pltpu.sync_copy(buf1, acc.at[dst_v.at[j + 1]], add=True)
                pltpu.async_copy(t.at[src_v.at[j + 3]], buf1, sem1)
                return carry2

            lax.fori_loop(0, IB // 2 - 1, step, 0)
            jlast = IB - 2
            pltpu.make_async_copy(t.at[src_v.at[jlast]], buf0, sem0).wait()
            pltpu.sync_copy(buf0, acc.at[dst_v.at[jlast]], add=True)
            pltpu.make_async_copy(t.at[src_v.at[jlast + 1]], buf1, sem1).wait()
            pltpu.sync_copy(buf1, acc.at[dst_v.at[jlast + 1]], add=True)
            return carry

        lax.fori_loop(0, nblk, block, 0)

        plsc.subcore_barrier()
        for hh in range(RPT // LANE):
            pltpu.sync_copy(acc.at[pl.ds(s * RPT + hh * LANE, LANE)], buf0)
            pltpu.sync_copy(
                buf0, aggp.at[pl.ds(c * Np + s * RPT + hh * LANE, LANE)])

    return pl.kernel(
        body,
        out_type=jax.ShapeDtypeStruct((2 * Np, H), jnp.float32),
        mesh=_mesh,
        scratch_types=[
            pltpu.VMEM((IB, LANE), jnp.int32),
            pltpu.VMEM((IB, LANE), jnp.int32),
            pltpu.VMEM((LANE, H), jnp.float32),
            pltpu.VMEM((LANE, H), jnp.float32),
            pltpu.VMEM_SHARED((Np, H), jnp.float32),
            pltpu.SemaphoreType.DMA,
            pltpu.SemaphoreType.DMA,
        ],
    )


def _prep_body(degp_ref, s2_ref):
    deg_out = jnp.maximum(degp_ref[0] + degp_ref[2], 1.0)
    deg_in = jnp.maximum(degp_ref[1] + degp_ref[3], 1.0)
    s2_ref[...] = jnp.concatenate(
        [lax.rsqrt(deg_out)[None], lax.rsqrt(deg_in)[None]], axis=0)


def _tc_prep(degp):
    return pl.pallas_call(
        _prep_body,
        out_shape=jax.ShapeDtypeStruct((2, Np), jnp.float32),
    )(degp)


def _mm_scale_body(x_ref, w_ref, so_ref, o_ref):
    o_ref[...] = jnp.dot(x_ref[...], w_ref[...],
                         preferred_element_type=jnp.float32) * so_ref[...]


def _tc_mm_scale(x, W1, so_col):
    grid = (N // ROWBLK,)
    return pl.pallas_call(
        _mm_scale_body,
        grid=grid,
        in_specs=[
            pl.BlockSpec((ROWBLK, D), lambda i: (i, 0)),
            pl.BlockSpec((D, H), lambda i: (0, 0)),
            pl.BlockSpec((ROWBLK, 1), lambda i: (i, 0)),
        ],
        out_specs=pl.BlockSpec((ROWBLK, H), lambda i: (i, 0)),
        out_shape=jax.ShapeDtypeStruct((N, H), jnp.float32),
    )(x, W1, so_col)


def _layer_body(p_ref, si_ref, so_ref, b_ref, w_ref, o_ref):
    h = jax.nn.relu((p_ref[0] + p_ref[1]) * si_ref[...] + b_ref[...])
    o_ref[...] = jnp.dot(h * so_ref[...], w_ref[...],
                         preferred_element_type=jnp.float32)


def _tc_layer(aggp, si_col, so_col, b1r, W2):
    grid = (N // ROWBLK,)
    return pl.pallas_call(
        _layer_body,
        grid=grid,
        in_specs=[
            pl.BlockSpec((2, ROWBLK, H), lambda i: (0, i, 0)),
            pl.BlockSpec((ROWBLK, 1), lambda i: (i, 0)),
            pl.BlockSpec((ROWBLK, 1), lambda i: (i, 0)),
            pl.BlockSpec((1, H), lambda i: (0, 0)),
            pl.BlockSpec((H, H), lambda i: (0, 0)),
        ],
        out_specs=pl.BlockSpec((ROWBLK, H), lambda i: (i, 0)),
        out_shape=jax.ShapeDtypeStruct((N, H), jnp.float32),
    )(aggp, si_col, so_col, b1r, W2)


def _final_body(p_ref, si_ref, b_ref, wc_ref, bc_ref, wr_ref, br_ref,
                o1_ref, o2_ref, acc_ref):
    i = pl.program_id(0)
    h = jax.nn.relu((p_ref[0] + p_ref[1]) * si_ref[...] + b_ref[...])
    ps = jnp.sum(h, axis=0, keepdims=True)

    @pl.when(i == 0)
    def _():
        acc_ref[...] = ps

    @pl.when(i > 0)
    def _():
        acc_ref[...] = acc_ref[...] + ps

    @pl.when(i == pl.num_programs(0) - 1)
    def _():
        hg = acc_ref[...] * (1.0 / N)
        o1_ref[...] = jnp.dot(hg, wc_ref[...],
                              preferred_element_type=jnp.float32) + bc_ref[...]
        o2_ref[...] = jnp.dot(hg, wr_ref[...],
                              preferred_element_type=jnp.float32) + br_ref[...]


def _tc_final(aggp, si_col, b2r, Wc, bcr, Wr, brr):
    grid = (N // ROWBLK,)
    nc = Wc.shape[1]
    nr = Wr.shape[1]
    return pl.pallas_call(
        _final_body,
        grid=grid,
        in_specs=[
            pl.BlockSpec((2, ROWBLK, H), lambda i: (0, i, 0)),
            pl.BlockSpec((ROWBLK, 1), lambda i: (i, 0)),
            pl.BlockSpec((1, H), lambda i: (0, 0)),
            pl.BlockSpec((H, nc), lambda i: (0, 0)),
            pl.BlockSpec((1, nc), lambda i: (0, 0)),
            pl.BlockSpec((H, nr), lambda i: (0, 0)),
            pl.BlockSpec((1, nr), lambda i: (0, 0)),
        ],
        out_specs=[
            pl.BlockSpec((1, nc), lambda i: (0, 0)),
            pl.BlockSpec((1, nr), lambda i: (0, 0)),
        ],
        out_shape=[
            jax.ShapeDtypeStruct((1, nc), jnp.float32),
            jax.ShapeDtypeStruct((1, nr), jnp.float32),
        ],
        scratch_shapes=[pltpu.VMEM((1, H), jnp.float32)],
    )(aggp, si_col, b2r, Wc, bcr, Wr, brr)


def kernel(x, edge_index, W1, b1, W2, b2, Wc, bc, Wr, br):
    E = edge_index.shape[1]
    per_w = -(-E // (NW * LANE)) * LANE
    ch = per_w // LANE
    if ch % 2:
        ch += 1
        per_w = ch * LANE
    e_pad = NW * per_w - E

    src = edge_index[0]
    dst = edge_index[1]
    pad_i = jnp.arange(e_pad, dtype=jnp.int32)
    pad_node = N + pad_i % (Np - N)
    srcg = jnp.concatenate([src, pad_i % N]).reshape(NW * ch, LANE)
    dstg = jnp.concatenate([dst, pad_node]).reshape(NW * ch, LANE)
    if E % LANE == 0:
        srcd = srcg
    else:
        srcd = jnp.concatenate([src, pad_node]).reshape(NW * ch, LANE)

    degp = _make_degrees(ch, E)(srcd, dstg)
    s2 = _tc_prep(degp.reshape(4, Np))
    so_col = s2[0].reshape(Np, 1)
    si_col = s2[1].reshape(Np, 1)

    segsum = _make_segsum(ch)
    t1 = _tc_mm_scale(x, W1, so_col)
    aggp1 = segsum(t1, srcg, dstg).reshape(2, Np, H)
    t2 = _tc_layer(aggp1, si_col, so_col, b1.reshape(1, H), W2)
    aggp2 = segsum(t2, srcg, dstg).reshape(2, Np, H)
    out_cat, out_cont = _tc_final(
        aggp2, si_col, b2.reshape(1, H), Wc, bc.reshape(1, Wc.shape[1]),
        Wr, br.reshape(1, Wr.shape[1]))
    return (out_cat, out_cont)
